# Initial kernel scaffold; baseline (speedup 1.0000x reference)
import jax
import jax.numpy as jnp
from jax.experimental import pallas as pl


def kernel(data, temporal_pos_enc, remain_idx, global_token):
    raise NotImplementedError("write your pallas kernel here")


# speedup vs baseline: 3.7899x; 3.7899x over previous
import functools

import jax
import jax.numpy as jnp
from jax import lax
from jax.experimental import pallas as pl
from jax.experimental.pallas import tpu as pltpu
from jax.experimental.pallas import tpu_sc as plsc

NC = 2
NS = 16
NW = NC * NS
L = 16

K = 32


def _build_sc_call(B, T, D, R):
    RPW = (B * R) // NW
    WPB = NW // B
    CH = RPW // K
    mesh = plsc.VectorSubcoreMesh(
        core_axis_name="c", subcore_axis_name="s",
        num_cores=NC, num_subcores=NS)

    @functools.partial(
        pl.kernel,
        mesh=mesh,
        compiler_params=pltpu.CompilerParams(use_tc_tiling_on_sc=False),
        out_type=jax.ShapeDtypeStruct((B * (R + 1), D), jnp.float32),
        scratch_types=[
            pltpu.VMEM((K,), jnp.int32),
            pltpu.VMEM((K,), jnp.int32),
            pltpu.VMEM((K,), jnp.int32),
            pltpu.VMEM((K, D), jnp.float32),
            pltpu.VMEM((K, D), jnp.float32),
            pltpu.VMEM((D,), jnp.float32),
            pltpu.VMEM((D,), jnp.float32),
            pltpu.SemaphoreType.DMA,
            pltpu.SemaphoreType.DMA,
        ],
    )
    def sc_call(data_hbm, pos_hbm, idx_hbm, gt_hbm, out_hbm,
                idx_v, didx_v, pidx_v, bufA, bufB, gtbuf, posbuf,
                semA, semB):
        w = lax.axis_index("s") * NC + lax.axis_index("c")
        b = w // WPB
        half = w % WPB
        g0 = w * RPW
        out_row0 = b * (R + 1) + 1 + half * RPW

        @pl.when(half == 0)
        def _():
            pltpu.sync_copy(gt_hbm.at[0], gtbuf)
            pltpu.sync_copy(pos_hbm.at[0], posbuf)
            for j in range(D // L):
                sl = pl.ds(j * L, L)
                gtbuf[sl] = gtbuf[sl] + posbuf[sl]
            pltpu.sync_copy(gtbuf, out_hbm.at[b * (R + 1)])

        def chunk_body(c, _):
            pltpu.sync_copy(idx_hbm.at[pl.ds(g0 + c * K, K)], idx_v)
            for t in range(K // L):
                sl = pl.ds(t * L, L)
                v = idx_v[sl]
                didx_v[sl] = v + b * T
                pidx_v[sl] = v + 1
            cpA = pltpu.async_copy(data_hbm.at[didx_v], bufA, semA)
            cpB = pltpu.async_copy(pos_hbm.at[pidx_v], bufB, semB)
            cpA.wait()
            cpB.wait()

            def row_body(i, _):
                for j in range(D // L):
                    sl = pl.ds(j * L, L)
                    bufA[i, sl] = bufA[i, sl] + bufB[i, sl]
                return _
            lax.fori_loop(0, K, row_body, None)

            pltpu.sync_copy(bufA, out_hbm.at[pl.ds(out_row0 + c * K, K)])
            return _

        lax.fori_loop(0, CH, chunk_body, None)

    return sc_call


@jax.jit
def kernel(data, temporal_pos_enc, remain_idx, global_token):
    B, T, D = data.shape
    R = remain_idx.shape[1]
    data_flat = data.reshape(B * T, D)
    pos = temporal_pos_enc[:T + 1]
    idx_flat = remain_idx.astype(jnp.int32).reshape(B * R)
    sc_call = _build_sc_call(B, T, D, R)
    out_flat = sc_call(data_flat, pos, idx_flat, global_token)
    return out_flat.reshape(B, R + 1, D)


# speedup vs baseline: 4.3965x; 1.1601x over previous
import functools

import jax
import jax.numpy as jnp
from jax import lax
from jax.experimental import pallas as pl
from jax.experimental.pallas import tpu as pltpu
from jax.experimental.pallas import tpu_sc as plsc

NC = 2
NS = 16
NW = NC * NS
L = 16

K = 16
NBUF = 2


def _build_sc_call(B, T, D, R):
    RPW = (B * R) // NW
    WPB = NW // B
    CH = RPW // K
    mesh = plsc.VectorSubcoreMesh(
        core_axis_name="c", subcore_axis_name="s",
        num_cores=NC, num_subcores=NS)

    @functools.partial(
        pl.kernel,
        mesh=mesh,
        compiler_params=pltpu.CompilerParams(use_tc_tiling_on_sc=False),
        out_type=jax.ShapeDtypeStruct((B * (R + 1), D), jnp.float32),
        scratch_types=[
            pltpu.VMEM((RPW,), jnp.int32),
            pltpu.VMEM((RPW,), jnp.int32),
            pltpu.VMEM((RPW,), jnp.int32),
            pltpu.VMEM((NBUF, K, D), jnp.float32),
            pltpu.VMEM((NBUF, K, D), jnp.float32),
            pltpu.VMEM((NBUF, K, D), jnp.float32),
            pltpu.VMEM((D,), jnp.float32),
            pltpu.VMEM((D,), jnp.float32),
            pltpu.SemaphoreType.DMA,
            pltpu.SemaphoreType.DMA,
            pltpu.SemaphoreType.DMA,
            pltpu.SemaphoreType.DMA,
            pltpu.SemaphoreType.DMA,
            pltpu.SemaphoreType.DMA,
        ],
    )
    def sc_call(data_hbm, pos_hbm, idx_hbm, gt_hbm, out_hbm,
                idx_all, didx, pidx, bufA, bufB, bufC, gtbuf, posbuf,
                sA0, sA1, sB0, sB1, sW0, sW1):
        sA = (sA0, sA1)
        sB = (sB0, sB1)
        sW = (sW0, sW1)
        w = lax.axis_index("s") * NC + lax.axis_index("c")
        b = w // WPB
        half = w % WPB
        g0 = w * RPW
        out_row0 = b * (R + 1) + 1 + half * RPW

        pltpu.sync_copy(idx_hbm.at[pl.ds(g0, RPW)], idx_all)

        def idx_body(t, carry):
            sl = pl.ds(t * L, L)
            v = idx_all[sl]
            didx[sl] = v + b * T
            pidx[sl] = v + 1
            return carry
        lax.fori_loop(0, RPW // L, idx_body, None)

        @pl.when(half == 0)
        def _():
            pltpu.sync_copy(gt_hbm.at[0], gtbuf)
            pltpu.sync_copy(pos_hbm.at[0], posbuf)
            for j in range(D // L):
                sl = pl.ds(j * L, L)
                gtbuf[sl] = gtbuf[sl] + posbuf[sl]
            pltpu.sync_copy(gtbuf, out_hbm.at[b * (R + 1)])

        def issue_gathers(c, s):
            pltpu.async_copy(
                data_hbm.at[didx.at[pl.ds(c * K, K)]], bufA.at[s], sA[s])
            pltpu.async_copy(
                pos_hbm.at[pidx.at[pl.ds(c * K, K)]], bufB.at[s], sB[s])

        for s in range(NBUF):
            issue_gathers(s, s)

        def outer(g, carry):
            for s in range(NBUF):
                c = g * NBUF + s
                pltpu.make_async_copy(
                    data_hbm.at[didx.at[pl.ds(c * K, K)]],
                    bufA.at[s], sA[s]).wait()
                pltpu.make_async_copy(
                    pos_hbm.at[pidx.at[pl.ds(c * K, K)]],
                    bufB.at[s], sB[s]).wait()

                @pl.when(g > 0)
                def _():
                    pltpu.make_async_copy(
                        bufC.at[s], out_hbm.at[pl.ds(out_row0, K)],
                        sW[s]).wait()

                def row_body(i, rc):
                    for j in range(D // L):
                        sl = pl.ds(j * L, L)
                        bufC[s, i, sl] = bufA[s, i, sl] + bufB[s, i, sl]
                    return rc
                lax.fori_loop(0, K, row_body, None)

                pltpu.async_copy(
                    bufC.at[s], out_hbm.at[pl.ds(out_row0 + c * K, K)],
                    sW[s])

                @pl.when(c + NBUF < CH)
                def _():
                    issue_gathers(c + NBUF, s)
            return carry
        lax.fori_loop(0, CH // NBUF, outer, None)

        for s in range(NBUF):
            pltpu.make_async_copy(
                bufC.at[s], out_hbm.at[pl.ds(out_row0, K)], sW[s]).wait()

    return sc_call


@jax.jit
def kernel(data, temporal_pos_enc, remain_idx, global_token):
    B, T, D = data.shape
    R = remain_idx.shape[1]
    data_flat = data.reshape(B * T, D)
    pos = temporal_pos_enc[:T + 1]
    idx_flat = remain_idx.astype(jnp.int32).reshape(B * R)
    sc_call = _build_sc_call(B, T, D, R)
    out_flat = sc_call(data_flat, pos, idx_flat, global_token)
    return out_flat.reshape(B, R + 1, D)


# speedup vs baseline: 6.1793x; 1.4055x over previous
import functools

import jax
import jax.numpy as jnp
from jax import lax
from jax.experimental import pallas as pl
from jax.experimental.pallas import tpu as pltpu
from jax.experimental.pallas import tpu_sc as plsc

NC = 2
NS = 16
NW = NC * NS
L = 16

K = 16
NBUF = 2


def _build_sc_call(B, T, D, R):
    RPW = (B * R) // NW
    WPB = NW // B
    CH = RPW // K
    mesh = plsc.VectorSubcoreMesh(
        core_axis_name="c", subcore_axis_name="s",
        num_cores=NC, num_subcores=NS)

    @functools.partial(
        pl.kernel,
        mesh=mesh,
        out_type=jax.ShapeDtypeStruct((B * (R + 1), D), jnp.float32),
        scratch_types=[
            pltpu.VMEM((RPW,), jnp.int32),
            pltpu.VMEM((RPW,), jnp.int32),
            pltpu.VMEM((RPW,), jnp.int32),
            pltpu.VMEM((NBUF, K, D), jnp.float32),
            pltpu.VMEM((NBUF, K, D), jnp.float32),
            pltpu.VMEM((NBUF, K, D), jnp.float32),
            pltpu.VMEM((K,), jnp.int32),
            pltpu.VMEM((K,), jnp.int32),
            pltpu.VMEM((L,), jnp.int32),
            pltpu.VMEM((L, D), jnp.float32),
            pltpu.VMEM((D,), jnp.float32),
            pltpu.VMEM((D,), jnp.float32),
            pltpu.SemaphoreType.DMA,
            pltpu.SemaphoreType.DMA,
            pltpu.SemaphoreType.DMA,
            pltpu.SemaphoreType.DMA,
            pltpu.SemaphoreType.DMA,
            pltpu.SemaphoreType.DMA,
            pltpu.SemaphoreType.DMA,
        ],
    )
    def sc_call(data_hbm, pos_hbm, idx_hbm, gt_hbm, out_hbm,
                idx_all, didx, pidx, bufA, bufB, bufC,
                oidx0, oidx1, gt_oidx, gtrows, gtbuf, posbuf,
                sA0, sA1, sB0, sB1, sW0, sW1, sG):
        sA = (sA0, sA1)
        sB = (sB0, sB1)
        sW = (sW0, sW1)
        oidx = (oidx0, oidx1)
        w = lax.axis_index("s") * NC + lax.axis_index("c")
        b = w // WPB
        half = w % WPB
        g0 = w * RPW
        out_row0 = b * (R + 1) + 1 + half * RPW

        pltpu.sync_copy(idx_hbm.at[pl.ds(g0, RPW)], idx_all)

        def idx_body(t, carry):
            sl = pl.ds(t * L, L)
            v = idx_all[sl]
            didx[sl] = v + b * T
            pidx[sl] = v + 1
            return carry
        lax.fori_loop(0, RPW // L, idx_body, None)

        @pl.when(w == 0)
        def _():
            pltpu.sync_copy(gt_hbm.at[0], gtbuf)
            pltpu.sync_copy(pos_hbm.at[0], posbuf)
            gt_oidx[pl.ds(0, L)] = lax.iota(jnp.int32, L) * (R + 1)

            def gt_body(i, carry):
                for j in range(D // L):
                    sl = pl.ds(j * L, L)
                    gtrows[i, sl] = gtbuf[sl] + posbuf[sl]
                return carry
            lax.fori_loop(0, L, gt_body, None)
            pltpu.async_copy(gtrows, out_hbm.at[gt_oidx], sG)

        def issue_gathers(c, s):
            pltpu.async_copy(
                data_hbm.at[didx.at[pl.ds(c * K, K)]], bufA.at[s], sA[s])
            pltpu.async_copy(
                pos_hbm.at[pidx.at[pl.ds(c * K, K)]], bufB.at[s], sB[s])

        for s in range(NBUF):
            issue_gathers(s, s)

        def outer(g, carry):
            for s in range(NBUF):
                c = g * NBUF + s
                pltpu.make_async_copy(
                    data_hbm.at[didx.at[pl.ds(c * K, K)]],
                    bufA.at[s], sA[s]).wait()
                pltpu.make_async_copy(
                    pos_hbm.at[pidx.at[pl.ds(c * K, K)]],
                    bufB.at[s], sB[s]).wait()

                @pl.when(g > 0)
                def _():
                    pltpu.make_async_copy(
                        bufC.at[s], out_hbm.at[oidx[s]], sW[s]).wait()

                def row_body(i, rc):
                    for j in range(D // L):
                        sl = pl.ds(j * L, L)
                        bufC[s, i, sl] = bufA[s, i, sl] + bufB[s, i, sl]
                    return rc
                lax.fori_loop(0, K, row_body, None)

                for t in range(K // L):
                    sl = pl.ds(t * L, L)
                    oidx[s][sl] = (out_row0 + c * K + t * L
                                   ) + lax.iota(jnp.int32, L)
                pltpu.async_copy(bufC.at[s], out_hbm.at[oidx[s]], sW[s])

                @pl.when(c + NBUF < CH)
                def _():
                    issue_gathers(c + NBUF, s)
            return carry
        lax.fori_loop(0, CH // NBUF, outer, None)

        for s in range(NBUF):
            pltpu.make_async_copy(
                bufC.at[s], out_hbm.at[oidx[s]], sW[s]).wait()

        @pl.when(w == 0)
        def _():
            pltpu.make_async_copy(gtrows, out_hbm.at[gt_oidx], sG).wait()

    return sc_call


@jax.jit
def kernel(data, temporal_pos_enc, remain_idx, global_token):
    B, T, D = data.shape
    R = remain_idx.shape[1]
    data_flat = data.reshape(B * T, D)
    pos = temporal_pos_enc[:T + 1]
    idx_flat = remain_idx.astype(jnp.int32).reshape(B * R)
    sc_call = _build_sc_call(B, T, D, R)
    out_flat = sc_call(data_flat, pos, idx_flat, global_token)
    return out_flat.reshape(B, R + 1, D)


# speedup vs baseline: 7.6673x; 1.2408x over previous
import functools

import jax
import jax.numpy as jnp
from jax import lax
from jax.experimental import pallas as pl
from jax.experimental.pallas import tpu as pltpu
from jax.experimental.pallas import tpu_sc as plsc

NC = 2
NS = 16
NW = NC * NS
L = 16

K = 16
NBUF = 2


def _build_sc_call(B, T, D, R):
    RPW = (B * R) // NW
    WPB = NW // B
    CH = RPW // K
    mesh = plsc.VectorSubcoreMesh(
        core_axis_name="c", subcore_axis_name="s",
        num_cores=NC, num_subcores=NS)

    @functools.partial(
        pl.kernel,
        mesh=mesh,
        out_type=jax.ShapeDtypeStruct((B, R + 1, D), jnp.float32),
        scratch_types=[
            pltpu.VMEM((RPW,), jnp.int32),
            pltpu.VMEM((RPW,), jnp.int32),
            pltpu.VMEM((RPW,), jnp.int32),
            pltpu.VMEM((NBUF, K, D), jnp.float32),
            pltpu.VMEM((NBUF, K, D), jnp.float32),
            pltpu.VMEM((NBUF, K, D), jnp.float32),
            pltpu.VMEM((K,), jnp.int32),
            pltpu.VMEM((K,), jnp.int32),
            pltpu.VMEM((L,), jnp.int32),
            pltpu.VMEM((L, D), jnp.float32),
            pltpu.VMEM((D,), jnp.float32),
            pltpu.VMEM((D,), jnp.float32),
            pltpu.SemaphoreType.DMA,
            pltpu.SemaphoreType.DMA,
            pltpu.SemaphoreType.DMA,
            pltpu.SemaphoreType.DMA,
            pltpu.SemaphoreType.DMA,
            pltpu.SemaphoreType.DMA,
            pltpu.SemaphoreType.DMA,
        ],
    )
    def sc_call(data_hbm, pos_hbm, idx_hbm, gt_hbm, out_hbm,
                idx_all, didx, pidx, bufA, bufB, bufC,
                oidx0, oidx1, gt_oidx, gtrows, gtbuf, posbuf,
                sA0, sA1, sB0, sB1, sW0, sW1, sG):
        sA = (sA0, sA1)
        sB = (sB0, sB1)
        sW = (sW0, sW1)
        oidx = (oidx0, oidx1)
        w = lax.axis_index("s") * NC + lax.axis_index("c")
        b = w // WPB
        half = w % WPB
        g0 = w * RPW
        out_row0 = 1 + half * RPW
        out_b = out_hbm.at[b]

        pltpu.sync_copy(idx_hbm.at[pl.ds(g0, RPW)], idx_all)

        def idx_body(t, carry):
            sl = pl.ds(t * L, L)
            v = idx_all[sl]
            didx[sl] = v + b * T
            pidx[sl] = v + 1
            return carry
        lax.fori_loop(0, RPW // L, idx_body, None)

        @pl.when(half == 0)
        def _():
            pltpu.sync_copy(gt_hbm.at[0], gtbuf)
            pltpu.sync_copy(pos_hbm.at[0], posbuf)
            gt_oidx[pl.ds(0, L)] = lax.iota(jnp.int32, L) * 0

            def gt_body(i, carry):
                for j in range(D // L):
                    sl = pl.ds(j * L, L)
                    gtrows[i, sl] = gtbuf[sl] + posbuf[sl]
                return carry
            lax.fori_loop(0, L, gt_body, None)
            pltpu.async_copy(gtrows, out_b.at[gt_oidx], sG)

        def issue_gathers(c, s):
            pltpu.async_copy(
                data_hbm.at[didx.at[pl.ds(c * K, K)]], bufA.at[s], sA[s])
            pltpu.async_copy(
                pos_hbm.at[pidx.at[pl.ds(c * K, K)]], bufB.at[s], sB[s])

        for s in range(NBUF):
            issue_gathers(s, s)

        def outer(g, carry):
            for s in range(NBUF):
                c = g * NBUF + s
                pltpu.make_async_copy(
                    data_hbm.at[didx.at[pl.ds(c * K, K)]],
                    bufA.at[s], sA[s]).wait()
                pltpu.make_async_copy(
                    pos_hbm.at[pidx.at[pl.ds(c * K, K)]],
                    bufB.at[s], sB[s]).wait()

                @pl.when(g > 0)
                def _():
                    pltpu.make_async_copy(
                        bufC.at[s], out_b.at[oidx[s]], sW[s]).wait()

                def row_body(i, rc):
                    for j in range(D // L):
                        sl = pl.ds(j * L, L)
                        bufC[s, i, sl] = bufA[s, i, sl] + bufB[s, i, sl]
                    return rc
                lax.fori_loop(0, K, row_body, None)

                for t in range(K // L):
                    sl = pl.ds(t * L, L)
                    oidx[s][sl] = (out_row0 + c * K + t * L
                                   ) + lax.iota(jnp.int32, L)
                pltpu.async_copy(bufC.at[s], out_b.at[oidx[s]], sW[s])

                @pl.when(c + NBUF < CH)
                def _():
                    issue_gathers(c + NBUF, s)
            return carry
        lax.fori_loop(0, CH // NBUF, outer, None)

        for s in range(NBUF):
            pltpu.make_async_copy(
                bufC.at[s], out_b.at[oidx[s]], sW[s]).wait()

        @pl.when(half == 0)
        def _():
            pltpu.make_async_copy(gtrows, out_b.at[gt_oidx], sG).wait()

    return sc_call


@jax.jit
def kernel(data, temporal_pos_enc, remain_idx, global_token):
    B, T, D = data.shape
    R = remain_idx.shape[1]
    data_flat = data.reshape(B * T, D)
    pos = temporal_pos_enc[:T + 1]
    idx_flat = remain_idx.astype(jnp.int32).reshape(B * R)
    sc_call = _build_sc_call(B, T, D, R)
    return sc_call(data_flat, pos, idx_flat, global_token)


# speedup vs baseline: 11.2611x; 1.4687x over previous
import functools

import jax
import jax.numpy as jnp
from jax import lax
from jax.experimental import pallas as pl
from jax.experimental.pallas import tpu as pltpu
from jax.experimental.pallas import tpu_sc as plsc

NC = 2
NS = 16
NW = NC * NS
L = 16

K = 16
NBUF = 2


def _build_sc_call(B, T, D, R):
    RPW = (B * R) // NW
    WPB = NW // B
    CH = RPW // K
    mesh = plsc.VectorSubcoreMesh(
        core_axis_name="c", subcore_axis_name="s",
        num_cores=NC, num_subcores=NS)

    @functools.partial(
        pl.kernel,
        mesh=mesh,
        out_type=jax.ShapeDtypeStruct(((R + 1) * B, D), jnp.float32),
        scratch_types=[
            pltpu.VMEM((RPW,), jnp.int32),
            pltpu.VMEM((RPW,), jnp.int32),
            pltpu.VMEM((RPW,), jnp.int32),
            pltpu.VMEM((NBUF, K, D), jnp.float32),
            pltpu.VMEM((NBUF, K, D), jnp.float32),
            pltpu.VMEM((NBUF, K, D), jnp.float32),
            pltpu.VMEM((K,), jnp.int32),
            pltpu.VMEM((K,), jnp.int32),
            pltpu.VMEM((L,), jnp.int32),
            pltpu.VMEM((L, D), jnp.float32),
            pltpu.VMEM((D,), jnp.float32),
            pltpu.VMEM((D,), jnp.float32),
            pltpu.SemaphoreType.DMA,
            pltpu.SemaphoreType.DMA,
            pltpu.SemaphoreType.DMA,
            pltpu.SemaphoreType.DMA,
            pltpu.SemaphoreType.DMA,
            pltpu.SemaphoreType.DMA,
            pltpu.SemaphoreType.DMA,
        ],
    )
    def sc_call(data_hbm, pos_hbm, idx_hbm, gt_hbm, out_hbm,
                idx_all, didx, pidx, bufA, bufB, bufC,
                oidx0, oidx1, gt_oidx, gtrows, gtbuf, posbuf,
                sA0, sA1, sB0, sB1, sW0, sW1, sG):
        sA = (sA0, sA1)
        sB = (sB0, sB1)
        sW = (sW0, sW1)
        oidx = (oidx0, oidx1)
        w = lax.axis_index("s") * NC + lax.axis_index("c")
        b = w // WPB
        half = w % WPB
        out_row0 = 1 + half * RPW

        pltpu.sync_copy(idx_hbm.at[b, pl.ds(half * RPW, RPW)], idx_all)

        def idx_body(t, carry):
            sl = pl.ds(t * L, L)
            v = idx_all[sl]
            didx[sl] = v + b * T
            pidx[sl] = v + 1
            return carry
        lax.fori_loop(0, RPW // L, idx_body, None)

        @pl.when(half == 0)
        def _():
            pltpu.sync_copy(gt_hbm.at[0], gtbuf)
            pltpu.sync_copy(pos_hbm.at[0], posbuf)
            gt_oidx[pl.ds(0, L)] = lax.iota(jnp.int32, L) * 0 + b

            def gt_body(i, carry):
                for j in range(D // L):
                    sl = pl.ds(j * L, L)
                    gtrows[i, sl] = gtbuf[sl] + posbuf[sl]
                return carry
            lax.fori_loop(0, L, gt_body, None)
            pltpu.async_copy(gtrows, out_hbm.at[gt_oidx], sG)

        def issue_gathers(c, s):
            pltpu.async_copy(
                data_hbm.at[didx.at[pl.ds(c * K, K)]], bufA.at[s], sA[s])
            pltpu.async_copy(
                pos_hbm.at[pidx.at[pl.ds(c * K, K)]], bufB.at[s], sB[s])

        for s in range(NBUF):
            issue_gathers(s, s)

        def outer(g, carry):
            for s in range(NBUF):
                c = g * NBUF + s
                pltpu.make_async_copy(
                    data_hbm.at[didx.at[pl.ds(c * K, K)]],
                    bufA.at[s], sA[s]).wait()
                pltpu.make_async_copy(
                    pos_hbm.at[pidx.at[pl.ds(c * K, K)]],
                    bufB.at[s], sB[s]).wait()

                @pl.when(g > 0)
                def _():
                    pltpu.make_async_copy(
                        bufC.at[s], out_hbm.at[oidx[s]], sW[s]).wait()

                def row_body(i, rc):
                    for j in range(D // L):
                        sl = pl.ds(j * L, L)
                        bufC[s, i, sl] = bufA[s, i, sl] + bufB[s, i, sl]
                    return rc
                lax.fori_loop(0, K, row_body, None)

                for t in range(K // L):
                    sl = pl.ds(t * L, L)
                    oidx[s][sl] = ((out_row0 + c * K + t * L) * B + b
                                   ) + lax.iota(jnp.int32, L) * B
                pltpu.async_copy(bufC.at[s], out_hbm.at[oidx[s]], sW[s])

                @pl.when(c + NBUF < CH)
                def _():
                    issue_gathers(c + NBUF, s)
            return carry
        lax.fori_loop(0, CH // NBUF, outer, None)

        for s in range(NBUF):
            pltpu.make_async_copy(
                bufC.at[s], out_hbm.at[oidx[s]], sW[s]).wait()

        @pl.when(half == 0)
        def _():
            pltpu.make_async_copy(gtrows, out_hbm.at[gt_oidx], sG).wait()

    return sc_call


@jax.jit
def kernel(data, temporal_pos_enc, remain_idx, global_token):
    B, T, D = data.shape
    R = remain_idx.shape[1]
    data_flat = data.reshape(B * T, D)
    pos = temporal_pos_enc[:T + 1]
    idx2d = remain_idx.astype(jnp.int32)
    sc_call = _build_sc_call(B, T, D, R)
    out_rb = sc_call(data_flat, pos, idx2d, global_token)
    return out_rb.reshape(R + 1, B, D).transpose(1, 0, 2)
